# baseline (speedup 1.0000x reference)
import jax
import jax.numpy as jnp
from jax import lax
from jax.experimental import pallas as pl
from jax.experimental.pallas import tpu as pltpu
from jax.experimental.pallas import tpu_sc as plsc

B = 16384
D = 16
PACK = 8
NC, NS = 2, 16
NW = NC * NS
BPW = B // NW
CHUNK = 128
NCH = BPW // CHUNK
PPW = BPW // PACK


def _gather_body(qu_h, qi_h, ou_h, oi_h, um_t, im_t, ug_t, ig_t,
                 um_o, im_o, gm_o,
                 qu_v, qi_v, ou_v, oi_v, ou_sh, oi_sh, ou_s, oi_s, pk0, pk1,
                 um_v, im_v, gm_v, sem0, sem1):
    sid = lax.axis_index("s")
    wid = sid * NC + lax.axis_index("c")
    base = wid * BPW
    in_sl = pl.ds(base, BPW)
    pltpu.sync_copy(qu_h.at[in_sl], qu_v)
    pltpu.sync_copy(qi_h.at[in_sl], qi_v)
    pltpu.sync_copy(ou_h.at[in_sl], ou_v)
    pltpu.sync_copy(oi_h.at[in_sl], oi_v)
    pltpu.sync_copy(ou_v, ou_sh.at[sid])
    pltpu.sync_copy(oi_v, oi_sh.at[sid])
    pltpu.sync_copy(ou_sh.at[sid], ou_s)
    pltpu.sync_copy(oi_sh.at[sid], oi_s)

    tables = (um_t, im_t, ug_t, ig_t)
    qids = (qu_v, qi_v, qu_v, qi_v)
    offs = (ou_s, oi_s, ou_s, oi_s)
    dsts = (um_v, im_v, gm_v, gm_v)
    bufs = (pk0, pk1)
    sems = (sem0, sem1)
    nslot = 4 * NCH

    def fire(s):
        t, c = s // NCH, s % NCH
        return pltpu.async_copy(
            tables[t].at[qids[t].at[pl.ds(c * CHUNK, CHUNK)]],
            bufs[s % 2], sems[s % 2])

    def extract(s):
        t, c = s // NCH, s % NCH
        buf, off_s, dst = bufs[s % 2], offs[t], dsts[t]

        def body(r, _):
            o = off_s[c * CHUNK + r]
            v = buf[r, pl.ds(o, D)]
            orow = c * (CHUNK // PACK) + lax.shift_right_logical(r, 3)
            ocol = lax.shift_left(lax.rem(r, PACK), 4)
            if t < 3:
                dst[orow, pl.ds(ocol, D)] = v
            else:
                dst[orow, pl.ds(ocol, D)] = dst[orow, pl.ds(ocol, D)] * v
            return 0

        lax.fori_loop(0, CHUNK, body, 0)

    cp = fire(0)
    for s in range(nslot):
        nxt = fire(s + 1) if s + 1 < nslot else None
        cp.wait()
        extract(s)
        cp = nxt

    out_sl = pl.ds(wid * PPW, PPW)
    pltpu.sync_copy(um_v, um_o.at[out_sl])
    pltpu.sync_copy(im_v, im_o.at[out_sl])
    pltpu.sync_copy(gm_v, gm_o.at[out_sl])


def _make_gather():
    return pl.kernel(
        _gather_body,
        out_type=tuple(
            jax.ShapeDtypeStruct((B // PACK, PACK * D), jnp.float32)
            for _ in range(3)),
        mesh=plsc.VectorSubcoreMesh(core_axis_name="c", subcore_axis_name="s"),
        scratch_types=[
            pltpu.VMEM((BPW,), jnp.int32),
            pltpu.VMEM((BPW,), jnp.int32),
            pltpu.VMEM((BPW,), jnp.int32),
            pltpu.VMEM((BPW,), jnp.int32),
            pltpu.VMEM_SHARED((NS, BPW), jnp.int32),
            pltpu.VMEM_SHARED((NS, BPW), jnp.int32),
            pltpu.SMEM((BPW,), jnp.int32),
            pltpu.SMEM((BPW,), jnp.int32),
            pltpu.VMEM((CHUNK, PACK * D), jnp.float32),
            pltpu.VMEM((CHUNK, PACK * D), jnp.float32),
            pltpu.VMEM((PPW, PACK * D), jnp.float32),
            pltpu.VMEM((PPW, PACK * D), jnp.float32),
            pltpu.VMEM((PPW, PACK * D), jnp.float32),
            pltpu.SemaphoreType.DMA,
            pltpu.SemaphoreType.DMA,
        ],
    )


NV = 1000000
LB = 4096


def _relayout_body(a, b, c, d, oa, ob, oc, od):
    for x, o in ((a, oa), (b, ob), (c, oc), (d, od)):
        o[...] = pltpu.einshape("a(bc)->b(ca)", x[...], c=PACK)


def _relayout(um, im, ug, ig):
    grid = (NV + LB - 1) // LB
    return pl.pallas_call(
        _relayout_body,
        grid=(grid,),
        in_specs=[pl.BlockSpec((D, LB), lambda i: (0, i))] * 4,
        out_specs=[pl.BlockSpec((LB // PACK, PACK * D), lambda i: (i, 0))] * 4,
        out_shape=[jax.ShapeDtypeStruct((NV // PACK, PACK * D), jnp.float32)] * 4,
    )(um.T, im.T, ug.T, ig.T)


def _mlp_body(um, im, gm, w1u, w1i, b1, w2, b2, wl, out):
    f32 = jnp.float32
    h = jnp.dot(um[...], w1u[...], preferred_element_type=f32)
    h = h + jnp.dot(im[...], w1i[...], preferred_element_type=f32)
    h = jax.nn.relu(h + b1[...])
    h2 = jax.nn.relu(
        jnp.dot(h, w2[...], preferred_element_type=f32) + b2[...])
    feats = jnp.concatenate([h2, gm[...]], axis=1)
    out[...] = jnp.dot(feats, wl[...], preferred_element_type=f32)


def kernel(user_indices, item_indices, user_emb_gmf, item_emb_gmf,
           user_emb_mlp, item_emb_mlp, W1, b1, W2, b2, Wl, bl):
    qu = lax.shift_right_logical(user_indices, 3)
    qi = lax.shift_right_logical(item_indices, 3)
    ou = lax.shift_left(user_indices & 7, 4)
    oi = lax.shift_left(item_indices & 7, 4)
    umq, imq, ugq, igq = _relayout(user_emb_mlp, item_emb_mlp,
                                   user_emb_gmf, item_emb_gmf)

    um, im, gm = _make_gather()(qu, qi, ou, oi, umq, imq, ugq, igq)

    eye = jnp.eye(PACK, dtype=jnp.float32)
    w1u = jnp.kron(eye, W1[:D])
    w1i = jnp.kron(eye, W1[D:])
    b1b = jnp.tile(b1, PACK).reshape(1, -1)
    w2b = jnp.kron(eye, W2)
    b2b = jnp.tile(b2, PACK).reshape(1, -1)
    wlb = jnp.concatenate([jnp.kron(eye, Wl[:D]), jnp.kron(eye, Wl[D:])],
                          axis=0)

    out_p = pl.pallas_call(
        _mlp_body,
        out_shape=jax.ShapeDtypeStruct((B // PACK, PACK), jnp.float32),
    )(um, im, gm, w1u, w1i, b1b, w2b, b2b, wlb)

    return out_p.reshape(B, 1) + bl


# speedup vs baseline: 3.4564x; 3.4564x over previous
import jax
import jax.numpy as jnp
from jax import lax
from jax.experimental import pallas as pl
from jax.experimental.pallas import tpu as pltpu
from jax.experimental.pallas import tpu_sc as plsc

B = 16384
D = 16
PACK = 8
NC, NS = 2, 16
NW = NC * NS
BPW = B // NW
CHUNK = 128
NCH = BPW // CHUNK
PPW = BPW // PACK


def _gather_body(qu_h, qi_h, ou_h, oi_h, um_t, im_t, ug_t, ig_t,
                 um_o, im_o, gm_o,
                 qu_v, qi_v, ou_v, oi_v, ou_sh, oi_sh, ou_s, oi_s, pk0, pk1,
                 um_v, im_v, gm_v, sem0, sem1):
    sid = lax.axis_index("s")
    wid = sid * NC + lax.axis_index("c")
    base = wid * BPW
    in_sl = pl.ds(base, BPW)
    pltpu.sync_copy(qu_h.at[in_sl], qu_v)
    pltpu.sync_copy(qi_h.at[in_sl], qi_v)
    pltpu.sync_copy(ou_h.at[in_sl], ou_v)
    pltpu.sync_copy(oi_h.at[in_sl], oi_v)
    pltpu.sync_copy(ou_v, ou_sh.at[sid])
    pltpu.sync_copy(oi_v, oi_sh.at[sid])
    pltpu.sync_copy(ou_sh.at[sid], ou_s)
    pltpu.sync_copy(oi_sh.at[sid], oi_s)

    tables = (um_t, im_t, ug_t, ig_t)
    qids = (qu_v, qi_v, qu_v, qi_v)
    offs = (ou_s, oi_s, ou_s, oi_s)
    dsts = (um_v, im_v, gm_v, gm_v)
    bufs = (pk0, pk1)
    sems = (sem0, sem1)
    nslot = 4 * NCH

    def fire(s):
        t, c = s // NCH, s % NCH
        return pltpu.async_copy(
            tables[t].at[qids[t].at[pl.ds(c * CHUNK, CHUNK)]],
            bufs[s % 2], sems[s % 2])

    def extract(s):
        t, c = s // NCH, s % NCH
        buf, off_s, dst = bufs[s % 2], offs[t], dsts[t]

        def body(r, _):
            o = off_s[c * CHUNK + r]
            v = buf[r, pl.ds(o, D)]
            orow = c * (CHUNK // PACK) + lax.shift_right_logical(r, 3)
            ocol = lax.shift_left(lax.rem(r, PACK), 4)
            if t < 3:
                dst[orow, pl.ds(ocol, D)] = v
            else:
                dst[orow, pl.ds(ocol, D)] = dst[orow, pl.ds(ocol, D)] * v
            return 0

        lax.fori_loop(0, CHUNK, body, 0)

    cp = fire(0)
    for s in range(nslot):
        nxt = fire(s + 1) if s + 1 < nslot else None
        cp.wait()
        extract(s)
        cp = nxt

    out_sl = pl.ds(wid * PPW, PPW)
    pltpu.sync_copy(um_v, um_o.at[out_sl])
    pltpu.sync_copy(im_v, im_o.at[out_sl])
    pltpu.sync_copy(gm_v, gm_o.at[out_sl])


def _make_gather():
    return pl.kernel(
        _gather_body,
        out_type=tuple(
            jax.ShapeDtypeStruct((B // PACK, PACK * D), jnp.float32)
            for _ in range(3)),
        mesh=plsc.VectorSubcoreMesh(core_axis_name="c", subcore_axis_name="s"),
        scratch_types=[
            pltpu.VMEM((BPW,), jnp.int32),
            pltpu.VMEM((BPW,), jnp.int32),
            pltpu.VMEM((BPW,), jnp.int32),
            pltpu.VMEM((BPW,), jnp.int32),
            pltpu.VMEM_SHARED((NS, BPW), jnp.int32),
            pltpu.VMEM_SHARED((NS, BPW), jnp.int32),
            pltpu.SMEM((BPW,), jnp.int32),
            pltpu.SMEM((BPW,), jnp.int32),
            pltpu.VMEM((CHUNK, PACK * D), jnp.float32),
            pltpu.VMEM((CHUNK, PACK * D), jnp.float32),
            pltpu.VMEM((PPW, PACK * D), jnp.float32),
            pltpu.VMEM((PPW, PACK * D), jnp.float32),
            pltpu.VMEM((PPW, PACK * D), jnp.float32),
            pltpu.SemaphoreType.DMA,
            pltpu.SemaphoreType.DMA,
        ],
    )


def _mlp_body(um, im, gm, w1u, w1i, b1, w2, b2, wl, out):
    f32 = jnp.float32
    h = jnp.dot(um[...], w1u[...], preferred_element_type=f32)
    h = h + jnp.dot(im[...], w1i[...], preferred_element_type=f32)
    h = jax.nn.relu(h + b1[...])
    h2 = jax.nn.relu(
        jnp.dot(h, w2[...], preferred_element_type=f32) + b2[...])
    feats = jnp.concatenate([h2, gm[...]], axis=1)
    out[...] = jnp.dot(feats, wl[...], preferred_element_type=f32)


def kernel(user_indices, item_indices, user_emb_gmf, item_emb_gmf,
           user_emb_mlp, item_emb_mlp, W1, b1, W2, b2, Wl, bl):
    qu = lax.shift_right_logical(user_indices, 3)
    qi = lax.shift_right_logical(item_indices, 3)
    ou = lax.shift_left(user_indices & 7, 4)
    oi = lax.shift_left(item_indices & 7, 4)
    def _pack(t):
        return jnp.transpose(t.T.reshape(D, -1, PACK), (1, 2, 0)).reshape(
            -1, PACK * D)

    umq = _pack(user_emb_mlp)
    imq = _pack(item_emb_mlp)
    ugq = _pack(user_emb_gmf)
    igq = _pack(item_emb_gmf)

    um, im, gm = _make_gather()(qu, qi, ou, oi, umq, imq, ugq, igq)

    eye = jnp.eye(PACK, dtype=jnp.float32)
    w1u = jnp.kron(eye, W1[:D])
    w1i = jnp.kron(eye, W1[D:])
    b1b = jnp.tile(b1, PACK).reshape(1, -1)
    w2b = jnp.kron(eye, W2)
    b2b = jnp.tile(b2, PACK).reshape(1, -1)
    wlb = jnp.concatenate([jnp.kron(eye, Wl[:D]), jnp.kron(eye, Wl[D:])],
                          axis=0)

    out_p = pl.pallas_call(
        _mlp_body,
        out_shape=jax.ShapeDtypeStruct((B // PACK, PACK), jnp.float32),
    )(um, im, gm, w1u, w1i, b1b, w2b, b2b, wlb)

    return out_p.reshape(B, 1) + bl
